# baseline (speedup 1.0000x reference)
import jax
import jax.numpy as jnp
from jax.experimental import pallas as pl
from jax.experimental.pallas import tpu as pltpu

_EPS = 1e-5


def _moments(v):
    g = jax.lax.dot_general(v, v, (((1,), (1,)), ((), ())),
                            preferred_element_type=jnp.float32)
    s = jnp.sum(v, axis=1, keepdims=True)
    return jnp.concatenate([g, jnp.broadcast_to(s, g.shape)], axis=1)


def _fold(ps, w, g_row, b_row, inv_m):
    C = w.shape[0]
    g, b = jnp.transpose(g_row), jnp.transpose(b_row)
    S, sv = ps[:, :C], ps[:, C:C + 1]
    mean = jnp.dot(w, sv, preferred_element_type=jnp.float32) * inv_m
    sq = jnp.sum(jnp.dot(w, S, preferred_element_type=jnp.float32) * w,
                 axis=1, keepdims=True)
    var = jnp.maximum(sq * inv_m - mean * mean, 0.0)
    scale = g * jax.lax.rsqrt(var + _EPS)
    return scale * w, b - mean * scale


def _make_stats1_body(B, C, HW):
    def _body(x_ref, o_ref):
        acc = None
        for b in range(B):
            m = _moments(x_ref[b].reshape(C, HW))
            acc = m if acc is None else acc + m
        o_ref[0] = acc
    return _body


def _make_stats2_body(B, C, HW, inv_m):
    def _body(w1_ref, g1_ref, b1_ref, ps1_ref, x_ref, o_ref):
        w1f, s1 = _fold(jnp.sum(ps1_ref[...], axis=0),
                        w1_ref[...], g1_ref[...], b1_ref[...], inv_m)
        acc = None
        for b in range(B):
            h = jnp.dot(w1f, x_ref[b].reshape(C, HW),
                        preferred_element_type=jnp.float32) + s1
            m = _moments(jnp.maximum(h, 0.0))
            acc = m if acc is None else acc + m
        o_ref[0] = acc
    return _body


def _make_apply_body(B, C, H, W, inv_m):
    def _body(w1_ref, g1_ref, b1_ref, w2_ref, g2_ref, b2_ref,
              ps1_ref, ps2_ref, x_ref, o_ref):
        w1f, s1 = _fold(jnp.sum(ps1_ref[...], axis=0),
                        w1_ref[...], g1_ref[...], b1_ref[...], inv_m)
        w2f, s2 = _fold(jnp.sum(ps2_ref[...], axis=0),
                        w2_ref[...], g2_ref[...], b2_ref[...], inv_m)
        for b in range(B):
            x = x_ref[b].reshape(C, H * W)
            r = jnp.maximum(jnp.dot(w1f, x,
                                    preferred_element_type=jnp.float32) + s1,
                            0.0)
            z = jnp.dot(w2f, r, preferred_element_type=jnp.float32) + s2
            out = (x * jax.nn.sigmoid(z)).astype(o_ref.dtype)
            o_ref[b] = out.reshape(C, H, W)
    return _body


def kernel(x, w1, g1, b1, w2, g2, b2):
    N, C, H, W = x.shape
    K = w1.shape[0]
    HW = H * W
    inv_m = 1.0 / (N * HW)

    B = 16 if N % 16 == 0 else 1
    G = N // B

    par = pltpu.CompilerParams(dimension_semantics=("parallel",))

    w1_f32 = w1.astype(jnp.float32)
    w2_f32 = w2.astype(jnp.float32)
    g1c, b1c = g1.astype(jnp.float32)[None, :], b1.astype(jnp.float32)[None, :]
    g2c, b2c = g2.astype(jnp.float32)[None, :], b2.astype(jnp.float32)[None, :]

    x_spec = pl.BlockSpec((B, C, H, W), lambda i: (i, 0, 0, 0))
    w_spec = pl.BlockSpec((K, C), lambda i: (0, 0))
    v_spec = pl.BlockSpec((1, K), lambda i: (0, 0))
    ps_spec = pl.BlockSpec((G, K, 2 * K), lambda i: (0, 0, 0))

    ps1 = pl.pallas_call(
        _make_stats1_body(B, C, HW),
        out_shape=jax.ShapeDtypeStruct((G, C, 2 * C), jnp.float32),
        grid=(G,),
        in_specs=[x_spec],
        out_specs=pl.BlockSpec((1, C, 2 * C), lambda i: (i, 0, 0)),
        compiler_params=par,
    )(x)

    ps2 = pl.pallas_call(
        _make_stats2_body(B, C, HW, inv_m),
        out_shape=jax.ShapeDtypeStruct((G, K, 2 * K), jnp.float32),
        grid=(G,),
        in_specs=[w_spec, v_spec, v_spec, ps_spec, x_spec],
        out_specs=pl.BlockSpec((1, K, 2 * K), lambda i: (i, 0, 0)),
        compiler_params=par,
    )(w1_f32, g1c, b1c, ps1, x)

    B3 = 8 if N % 8 == 0 else 1
    x3_spec = pl.BlockSpec((B3, C, H, W), lambda i: (i, 0, 0, 0))
    out = pl.pallas_call(
        _make_apply_body(B3, C, H, W, inv_m),
        out_shape=jax.ShapeDtypeStruct((N, C, H, W), x.dtype),
        grid=(N // B3,),
        in_specs=[w_spec, v_spec, v_spec, w_spec, v_spec, v_spec,
                  ps_spec, ps_spec, x3_spec],
        out_specs=x3_spec,
        compiler_params=par,
    )(w1_f32, g1c, b1c, w2_f32, g2c, b2c, ps1, ps2, x)

    return out


# speedup vs baseline: 1.0277x; 1.0277x over previous
import jax
import jax.numpy as jnp
from jax.experimental import pallas as pl
from jax.experimental.pallas import tpu as pltpu

_EPS = 1e-5


def _moments(v):
    g = jax.lax.dot_general(v, v, (((1,), (1,)), ((), ())),
                            preferred_element_type=jnp.float32)
    s = jnp.sum(v, axis=1, keepdims=True)
    return jnp.concatenate([g, jnp.broadcast_to(s, g.shape)], axis=1)


def _fold(ps, w, g_row, b_row, inv_m):
    C = w.shape[0]
    g, b = jnp.transpose(g_row), jnp.transpose(b_row)
    S, sv = ps[:, :C], ps[:, C:C + 1]
    mean = jnp.dot(w, sv, preferred_element_type=jnp.float32) * inv_m
    sq = jnp.sum(jnp.dot(w, S, preferred_element_type=jnp.float32) * w,
                 axis=1, keepdims=True)
    var = jnp.maximum(sq * inv_m - mean * mean, 0.0)
    scale = g * jax.lax.rsqrt(var + _EPS)
    return scale * w, b - mean * scale


def _make_stats1_body(B, C, HW):
    def _body(x_ref, o_ref, xb_ref):
        acc = None
        for b in range(B):
            v = x_ref[b].reshape(C, HW)
            m = _moments(v)
            acc = m if acc is None else acc + m
            xb_ref[b] = v.astype(jnp.bfloat16)
        o_ref[0] = acc
    return _body


def _make_stats2_body(B, C, HW, inv_m):
    def _body(w1_ref, g1_ref, b1_ref, ps1_ref, x_ref, o_ref):
        w1f, s1 = _fold(jnp.sum(ps1_ref[...], axis=0),
                        w1_ref[...], g1_ref[...], b1_ref[...], inv_m)
        acc = None
        for b in range(B):
            h = jnp.dot(w1f, x_ref[b].astype(jnp.float32),
                        preferred_element_type=jnp.float32) + s1
            m = _moments(jnp.maximum(h, 0.0))
            acc = m if acc is None else acc + m
        o_ref[0] = acc
    return _body


def _make_apply_body(B, C, H, W, inv_m):
    def _body(w1_ref, g1_ref, b1_ref, w2_ref, g2_ref, b2_ref,
              ps1_ref, ps2_ref, x_ref, o_ref):
        w1f, s1 = _fold(jnp.sum(ps1_ref[...], axis=0),
                        w1_ref[...], g1_ref[...], b1_ref[...], inv_m)
        w2f, s2 = _fold(jnp.sum(ps2_ref[...], axis=0),
                        w2_ref[...], g2_ref[...], b2_ref[...], inv_m)
        for b in range(B):
            x = x_ref[b].astype(jnp.float32)
            r = jnp.maximum(jnp.dot(w1f, x,
                                    preferred_element_type=jnp.float32) + s1,
                            0.0)
            z = jnp.dot(w2f, r, preferred_element_type=jnp.float32) + s2
            out = (x * jax.nn.sigmoid(z)).astype(o_ref.dtype)
            o_ref[b] = out.reshape(C, H, W)
    return _body


def kernel(x, w1, g1, b1, w2, g2, b2):
    N, C, H, W = x.shape
    K = w1.shape[0]
    HW = H * W
    inv_m = 1.0 / (N * HW)

    B = 8 if N % 8 == 0 else 1
    G = N // B

    par = pltpu.CompilerParams(dimension_semantics=("parallel",))

    w1_f32 = w1.astype(jnp.float32)
    w2_f32 = w2.astype(jnp.float32)
    g1c, b1c = g1.astype(jnp.float32)[None, :], b1.astype(jnp.float32)[None, :]
    g2c, b2c = g2.astype(jnp.float32)[None, :], b2.astype(jnp.float32)[None, :]

    x_spec = pl.BlockSpec((B, C, H, W), lambda i: (i, 0, 0, 0))
    w_spec = pl.BlockSpec((K, C), lambda i: (0, 0))
    v_spec = pl.BlockSpec((1, K), lambda i: (0, 0))
    ps_spec = pl.BlockSpec((G, K, 2 * K), lambda i: (0, 0, 0))

    xb_spec = pl.BlockSpec((B, C, HW), lambda i: (i, 0, 0))

    ps1, xb = pl.pallas_call(
        _make_stats1_body(B, C, HW),
        out_shape=(jax.ShapeDtypeStruct((G, C, 2 * C), jnp.float32),
                   jax.ShapeDtypeStruct((N, C, HW), jnp.bfloat16)),
        grid=(G,),
        in_specs=[x_spec],
        out_specs=(pl.BlockSpec((1, C, 2 * C), lambda i: (i, 0, 0)),
                   xb_spec),
        compiler_params=par,
    )(x)

    ps2 = pl.pallas_call(
        _make_stats2_body(B, C, HW, inv_m),
        out_shape=jax.ShapeDtypeStruct((G, K, 2 * K), jnp.float32),
        grid=(G,),
        in_specs=[w_spec, v_spec, v_spec, ps_spec, xb_spec],
        out_specs=pl.BlockSpec((1, K, 2 * K), lambda i: (i, 0, 0)),
        compiler_params=par,
    )(w1_f32, g1c, b1c, ps1, xb)

    out = pl.pallas_call(
        _make_apply_body(B, C, H, W, inv_m),
        out_shape=jax.ShapeDtypeStruct((N, C, H, W), x.dtype),
        grid=(G,),
        in_specs=[w_spec, v_spec, v_spec, w_spec, v_spec, v_spec,
                  ps_spec, ps_spec, xb_spec],
        out_specs=x_spec,
        compiler_params=par,
    )(w1_f32, g1c, b1c, w2_f32, g2c, b2c, ps1, ps2, xb)

    return out
